# baseline (speedup 1.0000x reference)
import functools

import jax
import jax.numpy as jnp
from jax import lax
from jax.experimental import pallas as pl
from jax.experimental.pallas import tpu as pltpu
from jax.experimental.pallas import tpu_sc as plsc

N = 100000
E = 100000
H = 128
RSZ = 4096
RSH = 12
NR = 25
NP = NR * RSZ
CAP = 5120
ACC = RSZ + 128
EP = 100352
TPW = EP // 32
NV = TPW // 16
TRG = CAP // 16
CH = 32
NCH = TRG // CH
ZR = 24
NB = 2048

_MESH = plsc.VectorSubcoreMesh(core_axis_name="c", subcore_axis_name="s")


def _init_body(x_ref, wiou_ref, biou_ref, wf_ref, bf_ref, uf_ref,
               ioux_ref, xwf_ref, chh_ref):
    x = x_ref[...]
    ioux = x @ wiou_ref[...] + biou_ref[...]
    ioux_ref[...] = ioux
    xwf_ref[...] = x @ wf_ref[...] + bf_ref[...]
    i = jax.nn.sigmoid(ioux[:, :H])
    o = jax.nn.sigmoid(ioux[:, H:2 * H])
    u = jnp.tanh(ioux[:, 2 * H:])
    c = i * u
    h = o * jnp.tanh(c)
    chh_ref[:, :H] = c
    chh_ref[:, H:2 * H] = h @ uf_ref[...]
    chh_ref[:, 2 * H:] = h


def _node_init(x, W_iou, b_iou, W_f, b_f, U_f):
    n = x.shape[0]
    return pl.pallas_call(
        _init_body,
        grid=(n // NB,),
        in_specs=[
            pl.BlockSpec((NB, H), lambda i: (i, 0)),
            pl.BlockSpec((H, 3 * H), lambda i: (0, 0)),
            pl.BlockSpec((3 * H,), lambda i: (0,)),
            pl.BlockSpec((H, H), lambda i: (0, 0)),
            pl.BlockSpec((H,), lambda i: (0,)),
            pl.BlockSpec((H, H), lambda i: (0, 0)),
        ],
        out_specs=[
            pl.BlockSpec((NB, 3 * H), lambda i: (i, 0)),
            pl.BlockSpec((NB, H), lambda i: (i, 0)),
            pl.BlockSpec((NB, 3 * H), lambda i: (i, 0)),
        ],
        out_shape=[
            jax.ShapeDtypeStruct((n, 3 * H), jnp.float32),
            jax.ShapeDtypeStruct((n, H), jnp.float32),
            jax.ShapeDtypeStruct((n, 3 * H), jnp.float32),
        ],
    )(x, W_iou, b_iou, W_f, b_f, U_f)


def _round_body(ioux_ref, hsum_ref, csum_ref, uiou_ref, uf_ref, chh_ref,
                *, last):
    iou = ioux_ref[...] + hsum_ref[...] @ uiou_ref[...]
    i = jax.nn.sigmoid(iou[:, :H])
    o = jax.nn.sigmoid(iou[:, H:2 * H])
    u = jnp.tanh(iou[:, 2 * H:])
    c = i * u + csum_ref[...]
    h = o * jnp.tanh(c)
    chh_ref[:, :H] = c
    if not last:
        chh_ref[:, H:2 * H] = h @ uf_ref[...]
    chh_ref[:, 2 * H:] = h


def _round_update(ioux, hsum, csum, U_iou, U_f, last=False):
    n = ioux.shape[0]
    return pl.pallas_call(
        functools.partial(_round_body, last=last),
        grid=(n // NB,),
        in_specs=[
            pl.BlockSpec((NB, 3 * H), lambda i: (i, 0)),
            pl.BlockSpec((NB, H), lambda i: (i, 0)),
            pl.BlockSpec((NB, H), lambda i: (i, 0)),
            pl.BlockSpec((H, 3 * H), lambda i: (0, 0)),
            pl.BlockSpec((H, H), lambda i: (0, 0)),
        ],
        out_specs=[
            pl.BlockSpec((NB, 3 * H), lambda i: (i, 0)),
        ],
        out_shape=[
            jax.ShapeDtypeStruct((n, 3 * H), jnp.float32),
        ],
    )(ioux, hsum, csum, U_iou, U_f)


def _offsets_body(cnt_ref, off_ref, tot_ref):
    c = cnt_ref[...].astype(jnp.float32)
    row = lax.broadcasted_iota(jnp.int32, (32, 32), 0)
    col = lax.broadcasted_iota(jnp.int32, (32, 32), 1)
    lstrict = jnp.where(col < row, 1.0, 0.0)
    excl = lstrict @ c
    rbase = col * CAP
    off_ref[...] = excl.astype(jnp.int32) + rbase
    tot_ref[...] = jnp.sum(cnt_ref[...], axis=0)


def _offsets(counts):
    return pl.pallas_call(
        _offsets_body,
        out_shape=[
            jax.ShapeDtypeStruct((32, 32), jnp.int32),
            jax.ShapeDtypeStruct((32,), jnp.int32),
        ],
    )(counts)


def _mlp_body(ha_ref, hb_ref, whw_ref, whb_ref, wpw_ref, wpb_ref, r_ref,
              out_ref, pred_ref):
    ha = ha_ref[...]
    hb = hb_ref[...]
    vec = jnp.concatenate([ha * hb, jnp.abs(ha - hb)], axis=1)
    hid = jax.nn.sigmoid(vec @ whw_ref[...] + whb_ref[...])
    logits = hid @ wpw_ref[...] + wpb_ref[...]
    out = jax.nn.log_softmax(logits, axis=1)
    out_ref[...] = out
    pred_ref[...] = jnp.exp(out) @ r_ref[...]


def _final_mlp(ha, hb, wh_W, wh_b, wp_W, wp_b, r):
    R = ha.shape[0]
    C = wp_W.shape[1]
    return pl.pallas_call(
        _mlp_body,
        out_shape=[
            jax.ShapeDtypeStruct((R, C), jnp.float32),
            jax.ShapeDtypeStruct((R,), jnp.float32),
        ],
    )(ha, hb, wh_W, wh_b, wp_W, wp_b, r)


_GDN = lax.GatherDimensionNumbers(
    offset_dims=(), collapsed_slice_dims=(0,), start_index_map=(0,))


def _lane_gather(v, idx):
    return lax.gather(v, idx[:, None], _GDN, slice_sizes=(1,),
                      mode=lax.GatherScatterMode.PROMISE_IN_BOUNDS)


def _vsum_splat(v):
    lanes = lax.iota(jnp.int32, 16)
    for k in (1, 2, 4, 8):
        v = v + _lane_gather(v, jnp.bitwise_and(lanes + k, 15))
    return v


def _incl_cumsum(v):
    lanes = lax.iota(jnp.int32, 16)
    incl = v
    for k in (1, 2, 4, 8):
        sh = _lane_gather(incl, jnp.maximum(lanes - k, 0))
        incl = incl + jnp.where(lanes >= k, sh, 0)
    return incl

def _sc_gather(table, idx, chunk):
    B = idx.shape[0]
    D = table.shape[1]
    per_w = B // 32
    nch = per_w // chunk

    @functools.partial(
        pl.kernel,
        out_type=jax.ShapeDtypeStruct((B, D), jnp.float32),
        mesh=_MESH,
        scratch_types=[
            pltpu.VMEM((per_w,), jnp.int32),
            pltpu.VMEM((chunk, D), jnp.float32),
            pltpu.SemaphoreType.DMA,
        ],
    )
    def k(table_hbm, idx_hbm, out_hbm, idx_v, rows_v, sem):
        w = lax.axis_index("s") * 2 + lax.axis_index("c")
        base = w * per_w
        pltpu.sync_copy(idx_hbm.at[pl.ds(base, per_w)], idx_v)
        for j in range(nch):
            pltpu.async_copy(
                table_hbm.at[idx_v.at[pl.ds(j * chunk, chunk)]], rows_v,
                sem).wait()
            pltpu.sync_copy(rows_v,
                            out_hbm.at[pl.ds(base + j * chunk, chunk)])

    return k(table, idx)


@functools.partial(
    pl.kernel,
    out_type=jax.ShapeDtypeStruct((32, 32), jnp.int32),
    mesh=_MESH,
    scratch_types=[
        pltpu.VMEM((TPW,), jnp.int32),
        pltpu.VMEM((32,), jnp.int32),
    ],
)
def _b1_count(dst_hbm, counts_hbm, dbuf, cbuf):
    w = lax.axis_index("s") * 2 + lax.axis_index("c")
    pltpu.sync_copy(dst_hbm.at[pl.ds(w * TPW, TPW)], dbuf)
    lanes = lax.iota(jnp.int32, 16)
    zero16 = jnp.zeros((16,), jnp.int32)

    def body(i, carry):
        c0, c1 = carry
        d = dbuf[pl.ds(i * 16, 16)]
        r = lax.shift_right_logical(d, RSH)
        for rr in range(NR):
            mi = jnp.where(r == rr, 1, 0).astype(jnp.int32)
            pc = _vsum_splat(mi)
            if rr < 16:
                c0 = c0 + jnp.where(lanes == rr, pc, 0)
            else:
                c1 = c1 + jnp.where(lanes == rr - 16, pc, 0)
        return c0, c1

    c0, c1 = lax.fori_loop(0, NV, body, (zero16, zero16))
    cbuf[pl.ds(0, 16)] = c0
    cbuf[pl.ds(16, 16)] = c1
    pltpu.sync_copy(cbuf, counts_hbm.at[w])


@functools.partial(
    pl.kernel,
    out_type=[
        jax.ShapeDtypeStruct((NR * CAP,), jnp.int32),
        jax.ShapeDtypeStruct((NR * CAP,), jnp.int32),
    ],
    mesh=_MESH,
    scratch_types=[
        pltpu.VMEM((TPW,), jnp.int32),
        pltpu.VMEM((TPW,), jnp.int32),
        pltpu.VMEM((TPW,), jnp.int32),
        pltpu.VMEM((TPW,), jnp.int32),
        pltpu.VMEM((32,), jnp.int32),
    ],
)
def _b2_place(dst_hbm, src_hbm, offs_hbm, srcb_hbm, dstob_hbm,
              dbuf, sbuf, posb, dlob, obuf):
    w = lax.axis_index("s") * 2 + lax.axis_index("c")
    base = w * TPW
    pltpu.sync_copy(dst_hbm.at[pl.ds(base, TPW)], dbuf)
    pltpu.sync_copy(src_hbm.at[pl.ds(base, TPW)], sbuf)
    pltpu.sync_copy(offs_hbm.at[w], obuf)
    o0 = obuf[pl.ds(0, 16)]
    o1 = obuf[pl.ds(16, 16)]
    offs0 = tuple(o0[rr] if rr < 16 else o1[rr - 16] for rr in range(NR))

    def body(i, offs):
        offs = list(offs)
        d = dbuf[pl.ds(i * 16, 16)]
        r = lax.shift_right_logical(d, RSH)
        pos = jnp.zeros((16,), jnp.int32)
        for rr in range(NR):
            m = r == rr
            mi = jnp.where(m, 1, 0).astype(jnp.int32)
            incl = _incl_cumsum(mi)
            pos = jnp.where(m, offs[rr] + incl - mi, pos)
            offs[rr] = offs[rr] + incl[15]
        posb[pl.ds(i * 16, 16)] = pos
        dlob[pl.ds(i * 16, 16)] = jnp.bitwise_and(d, RSZ - 1)
        return tuple(offs)

    lax.fori_loop(0, NV, body, offs0)
    pltpu.sync_copy(sbuf, srcb_hbm.at[posb])
    pltpu.sync_copy(dlob, dstob_hbm.at[posb])


_RND_SCRATCH = (
    [pltpu.VMEM_SHARED((ACC, H), jnp.float32)] * 2 +
    [pltpu.VMEM((ZR, H), jnp.float32)] +
    [pltpu.VMEM((CH, 3 * H), jnp.float32)] * 2 +
    [pltpu.VMEM((CH, H), jnp.float32)] * 2 +
    [pltpu.VMEM((CH, H), jnp.float32)] * 2 +
    [pltpu.VMEM((CH,), jnp.int32)] * 6 +
    [pltpu.VMEM((32,), jnp.int32)] +
    [pltpu.SemaphoreType.DMA] * 4
)


@functools.partial(
    pl.kernel,
    out_type=[
        jax.ShapeDtypeStruct((NP, H), jnp.float32),
        jax.ShapeDtypeStruct((NP, H), jnp.float32),
    ],
    mesh=_MESH,
    scratch_types=_RND_SCRATCH,
)
def _sc_round(chh_hbm, xwf_hbm, srcb_hbm, dstob_hbm, tot_hbm,
              hsum_hbm, csum_hbm,
              acc_h, acc_c, zbuf,
              rn0, rn1, rs0, rs1, rx0, rx1,
              sidx0, didx0, dlo0, sidx1, didx1, dlo1, tbuf,
              s0, s1, s2, s3):
    cid = lax.axis_index("c")
    sid = lax.axis_index("s")
    lanes = lax.iota(jnp.int32, 16)
    pltpu.sync_copy(tot_hbm, tbuf)
    t0 = tbuf[pl.ds(0, 16)]
    t1 = tbuf[pl.ds(16, 16)]

    bufs = [
        (rn0, rs0, rx0, sidx0, didx0, dlo0, (s0, s1)),
        (rn1, rs1, rx1, sidx1, didx1, dlo1, (s2, s3)),
    ]

    zrows = ACC // 16

    def zrow(i, _):
        row = i // 8
        col = (i - row * 8) * 16
        zbuf[row, pl.ds(col, 16)] = jnp.zeros((16,), jnp.float32)
        return 0

    lax.fori_loop(0, ZR * 8, zrow, 0)

    def start_chunk(j, r, total):
        rn, _, rx, sidx, didx, dlo, sems = bufs[j % 2]
        pos0 = r * CAP + sid * TRG + j * CH
        pltpu.sync_copy(srcb_hbm.at[pl.ds(pos0, CH)], sidx)
        pltpu.sync_copy(dstob_hbm.at[pl.ds(pos0, CH)], dlo)

        def idx_prep(m, _):
            s = sidx[pl.ds(m * 16, 16)]
            d = dlo[pl.ds(m * 16, 16)]
            gpos = sid * TRG + j * CH + m * 16 + lanes
            valid = gpos < total
            spread = jnp.bitwise_and(gpos, 63)
            s = jnp.where(valid, s, spread)
            d = jnp.where(valid, d, RSZ + spread)
            sidx[pl.ds(m * 16, 16)] = s
            dlo[pl.ds(m * 16, 16)] = d
            didx[pl.ds(m * 16, 16)] = jnp.minimum(r * RSZ + d, NP - 1)
            return 0

        lax.fori_loop(0, CH // 16, idx_prep, 0)
        return (pltpu.async_copy(chh_hbm.at[sidx], rn, sems[0]),
                pltpu.async_copy(xwf_hbm.at[didx], rx, sems[1]))

    def finish_chunk(j, cps):
        rn, rs, rx, _, _, dlo, _ = bufs[j % 2]
        for cp in cps:
            cp.wait()

        def fc_body(q, _):
            row = q // 8
            col = (q - row * 8) * 16
            xv = rx[row, pl.ds(col, 16)]
            hv = rn[row, pl.ds(H + col, 16)]
            cv = rn[row, pl.ds(col, 16)]
            f = 1.0 / (1.0 + jnp.exp(-(xv + hv)))
            rx[row, pl.ds(col, 16)] = f * cv
            rs[row, pl.ds(col, 16)] = rn[row, pl.ds(2 * H + col, 16)]
            return 0

        lax.fori_loop(0, CH * 8, fc_body, 0)
        pltpu.sync_copy(rs, acc_h.at[dlo], add=True)
        pltpu.sync_copy(rx, acc_c.at[dlo], add=True)

    def range_body(k, _):
        r = 2 * k + cid

        @pl.when(r < NR)
        def _():
            tv = (jnp.where(lanes == r, t0, 0) +
                  jnp.where(lanes == r - 16, t1, 0))
            total = _vsum_splat(tv)
            for z in range(zrows // ZR):
                pltpu.sync_copy(
                    zbuf, acc_h.at[pl.ds(sid * zrows + z * ZR, ZR)])
                pltpu.sync_copy(
                    zbuf, acc_c.at[pl.ds(sid * zrows + z * ZR, ZR)])
            plsc.subcore_barrier()
            cps = start_chunk(0, r, total)
            for j in range(NCH):
                nxt = (start_chunk(j + 1, r, total)
                       if j + 1 < NCH else None)
                finish_chunk(j, cps)
                cps = nxt
            plsc.subcore_barrier()
            wr = RSZ // 16
            pltpu.sync_copy(acc_h.at[pl.ds(sid * wr, wr)],
                            hsum_hbm.at[pl.ds(r * RSZ + sid * wr, wr)])
            pltpu.sync_copy(acc_c.at[pl.ds(sid * wr, wr)],
                            csum_hbm.at[pl.ds(r * RSZ + sid * wr, wr)])
            plsc.subcore_barrier()

        return 0

    lax.fori_loop(0, 13, range_body, 0)


def _tree_setup(wordid, edge_index, emb, W_iou, U_iou, b_iou, W_f, U_f,
                b_f):
    pad_n = NP - N
    wid = jnp.concatenate(
        [wordid.astype(jnp.int32),
         (jnp.arange(pad_n, dtype=jnp.int32) & 63)])
    x = _sc_gather(emb, wid, 320)
    src = edge_index[0].astype(jnp.int32)
    dst = edge_index[1].astype(jnp.int32)
    pad_e = EP - E
    src_p = jnp.concatenate(
        [src, (jnp.arange(pad_e, dtype=jnp.int32) & 63)])
    dst_p = jnp.concatenate(
        [dst, jnp.full((pad_e,), NP - 1, jnp.int32)])
    counts = _b1_count(dst_p)
    offs, totals = _offsets(counts)
    srcb, dstob = _b2_place(dst_p, src_p, offs)
    ioux, xwf, chh = _node_init(x, W_iou, b_iou, W_f, b_f, U_f)
    return ioux, xwf, chh, srcb, dstob, totals


def kernel(wordid_a, edge_index_a, root_ids_a, wordid_b, edge_index_b,
           root_ids_b, emb, W_iou, U_iou, b_iou, W_f, U_f, b_f, wh_W, wh_b,
           wp_W, wp_b, r):
    iouxa, xwfa, chha, srcba, dstoba, tota = _tree_setup(
        wordid_a, edge_index_a, emb, W_iou, U_iou, b_iou, W_f, U_f, b_f)
    iouxb, xwfb, chhb, srcbb, dstobb, totb = _tree_setup(
        wordid_b, edge_index_b, emb, W_iou, U_iou, b_iou, W_f, U_f, b_f)
    for k in range(3):
        hsa, csa = _sc_round(chha, xwfa, srcba, dstoba, tota)
        hsb, csb = _sc_round(chhb, xwfb, srcbb, dstobb, totb)
        chha = _round_update(iouxa, hsa, csa, U_iou, U_f, last=(k == 2))[0]
        chhb = _round_update(iouxb, hsb, csb, U_iou, U_f, last=(k == 2))[0]
    ha = _sc_gather(chha, root_ids_a.astype(jnp.int32), 16)[:, 2 * H:]
    hb = _sc_gather(chhb, root_ids_b.astype(jnp.int32), 16)[:, 2 * H:]
    out, pred = _final_mlp(ha, hb, wh_W, wh_b, wp_W, wp_b, r)
    return (out, pred)


# speedup vs baseline: 1.2291x; 1.2291x over previous
import functools

import jax
import jax.numpy as jnp
from jax import lax
from jax.experimental import pallas as pl
from jax.experimental.pallas import tpu as pltpu
from jax.experimental.pallas import tpu_sc as plsc

N = 100000
E = 100000
H = 128
RSZ = 4096
RSH = 12
NR = 25
NP = NR * RSZ
CAP = 5120
ACC = RSZ + 64
EP = 100352
TPW = EP // 32
NV = TPW // 16
TRG = CAP // 16
CH = 32
NCH = TRG // CH
ZR = 65
NB = 2048

_MESH = plsc.VectorSubcoreMesh(core_axis_name="c", subcore_axis_name="s")


def _init_body(x_ref, wiou_ref, biou_ref, wf_ref, bf_ref, uf_ref,
               ioux_ref, xwf_ref, h_ref, c_ref, huf_ref):
    x = x_ref[...]
    ioux = x @ wiou_ref[...] + biou_ref[...]
    ioux_ref[...] = ioux
    xwf_ref[...] = x @ wf_ref[...] + bf_ref[...]
    i = jax.nn.sigmoid(ioux[:, :H])
    o = jax.nn.sigmoid(ioux[:, H:2 * H])
    u = jnp.tanh(ioux[:, 2 * H:])
    c = i * u
    h = o * jnp.tanh(c)
    c_ref[...] = c
    h_ref[...] = h
    huf_ref[...] = h @ uf_ref[...]


def _node_init(x, W_iou, b_iou, W_f, b_f, U_f):
    n = x.shape[0]
    return pl.pallas_call(
        _init_body,
        grid=(n // NB,),
        in_specs=[
            pl.BlockSpec((NB, H), lambda i: (i, 0)),
            pl.BlockSpec((H, 3 * H), lambda i: (0, 0)),
            pl.BlockSpec((3 * H,), lambda i: (0,)),
            pl.BlockSpec((H, H), lambda i: (0, 0)),
            pl.BlockSpec((H,), lambda i: (0,)),
            pl.BlockSpec((H, H), lambda i: (0, 0)),
        ],
        out_specs=[
            pl.BlockSpec((NB, 3 * H), lambda i: (i, 0)),
            pl.BlockSpec((NB, H), lambda i: (i, 0)),
            pl.BlockSpec((NB, H), lambda i: (i, 0)),
            pl.BlockSpec((NB, H), lambda i: (i, 0)),
            pl.BlockSpec((NB, H), lambda i: (i, 0)),
        ],
        out_shape=[
            jax.ShapeDtypeStruct((n, 3 * H), jnp.float32),
            jax.ShapeDtypeStruct((n, H), jnp.float32),
            jax.ShapeDtypeStruct((n, H), jnp.float32),
            jax.ShapeDtypeStruct((n, H), jnp.float32),
            jax.ShapeDtypeStruct((n, H), jnp.float32),
        ],
    )(x, W_iou, b_iou, W_f, b_f, U_f)


def _round_body(ioux_ref, hsum_ref, csum_ref, uiou_ref, uf_ref,
                h_ref, c_ref, huf_ref, *, last):
    iou = ioux_ref[...] + hsum_ref[...] @ uiou_ref[...]
    i = jax.nn.sigmoid(iou[:, :H])
    o = jax.nn.sigmoid(iou[:, H:2 * H])
    u = jnp.tanh(iou[:, 2 * H:])
    c = i * u + csum_ref[...]
    h = o * jnp.tanh(c)
    c_ref[...] = c
    h_ref[...] = h
    if not last:
        huf_ref[...] = h @ uf_ref[...]


def _round_update(ioux, hsum, csum, U_iou, U_f, last=False):
    n = ioux.shape[0]
    return pl.pallas_call(
        functools.partial(_round_body, last=last),
        grid=(n // NB,),
        in_specs=[
            pl.BlockSpec((NB, 3 * H), lambda i: (i, 0)),
            pl.BlockSpec((NB, H), lambda i: (i, 0)),
            pl.BlockSpec((NB, H), lambda i: (i, 0)),
            pl.BlockSpec((H, 3 * H), lambda i: (0, 0)),
            pl.BlockSpec((H, H), lambda i: (0, 0)),
        ],
        out_specs=[
            pl.BlockSpec((NB, H), lambda i: (i, 0)),
            pl.BlockSpec((NB, H), lambda i: (i, 0)),
            pl.BlockSpec((NB, H), lambda i: (i, 0)),
        ],
        out_shape=[
            jax.ShapeDtypeStruct((n, H), jnp.float32),
            jax.ShapeDtypeStruct((n, H), jnp.float32),
            jax.ShapeDtypeStruct((n, H), jnp.float32),
        ],
    )(ioux, hsum, csum, U_iou, U_f)


def _offsets_body(cnt_ref, off_ref, tot_ref):
    c = cnt_ref[...].astype(jnp.float32)
    row = lax.broadcasted_iota(jnp.int32, (32, 32), 0)
    col = lax.broadcasted_iota(jnp.int32, (32, 32), 1)
    lstrict = jnp.where(col < row, 1.0, 0.0)
    excl = lstrict @ c
    rbase = col * CAP
    off_ref[...] = excl.astype(jnp.int32) + rbase
    tot_ref[...] = jnp.sum(cnt_ref[...], axis=0)


def _offsets(counts):
    return pl.pallas_call(
        _offsets_body,
        out_shape=[
            jax.ShapeDtypeStruct((32, 32), jnp.int32),
            jax.ShapeDtypeStruct((32,), jnp.int32),
        ],
    )(counts)


def _mlp_body(ha_ref, hb_ref, whw_ref, whb_ref, wpw_ref, wpb_ref, r_ref,
              out_ref, pred_ref):
    ha = ha_ref[...]
    hb = hb_ref[...]
    vec = jnp.concatenate([ha * hb, jnp.abs(ha - hb)], axis=1)
    hid = jax.nn.sigmoid(vec @ whw_ref[...] + whb_ref[...])
    logits = hid @ wpw_ref[...] + wpb_ref[...]
    out = jax.nn.log_softmax(logits, axis=1)
    out_ref[...] = out
    pred_ref[...] = jnp.exp(out) @ r_ref[...]


def _final_mlp(ha, hb, wh_W, wh_b, wp_W, wp_b, r):
    R = ha.shape[0]
    C = wp_W.shape[1]
    return pl.pallas_call(
        _mlp_body,
        out_shape=[
            jax.ShapeDtypeStruct((R, C), jnp.float32),
            jax.ShapeDtypeStruct((R,), jnp.float32),
        ],
    )(ha, hb, wh_W, wh_b, wp_W, wp_b, r)


_GDN = lax.GatherDimensionNumbers(
    offset_dims=(), collapsed_slice_dims=(0,), start_index_map=(0,))


def _lane_gather(v, idx):
    return lax.gather(v, idx[:, None], _GDN, slice_sizes=(1,),
                      mode=lax.GatherScatterMode.PROMISE_IN_BOUNDS)


def _vsum_splat(v):
    lanes = lax.iota(jnp.int32, 16)
    for k in (1, 2, 4, 8):
        v = v + _lane_gather(v, jnp.bitwise_and(lanes + k, 15))
    return v


def _incl_cumsum(v):
    lanes = lax.iota(jnp.int32, 16)
    incl = v
    for k in (1, 2, 4, 8):
        sh = _lane_gather(incl, jnp.maximum(lanes - k, 0))
        incl = incl + jnp.where(lanes >= k, sh, 0)
    return incl

def _sc_gather(table, idx, chunk):
    B = idx.shape[0]
    D = table.shape[1]
    per_w = B // 32
    nch = per_w // chunk

    @functools.partial(
        pl.kernel,
        out_type=jax.ShapeDtypeStruct((B, D), jnp.float32),
        mesh=_MESH,
        scratch_types=[
            pltpu.VMEM((per_w,), jnp.int32),
            pltpu.VMEM((chunk, D), jnp.float32),
            pltpu.SemaphoreType.DMA,
        ],
    )
    def k(table_hbm, idx_hbm, out_hbm, idx_v, rows_v, sem):
        w = lax.axis_index("s") * 2 + lax.axis_index("c")
        base = w * per_w
        pltpu.sync_copy(idx_hbm.at[pl.ds(base, per_w)], idx_v)
        for j in range(nch):
            pltpu.async_copy(
                table_hbm.at[idx_v.at[pl.ds(j * chunk, chunk)]], rows_v,
                sem).wait()
            pltpu.sync_copy(rows_v,
                            out_hbm.at[pl.ds(base + j * chunk, chunk)])

    return k(table, idx)


@functools.partial(
    pl.kernel,
    out_type=jax.ShapeDtypeStruct((32, 32), jnp.int32),
    mesh=_MESH,
    scratch_types=[
        pltpu.VMEM((TPW,), jnp.int32),
        pltpu.VMEM((32,), jnp.int32),
    ],
)
def _b1_count(dst_hbm, counts_hbm, dbuf, cbuf):
    w = lax.axis_index("s") * 2 + lax.axis_index("c")
    pltpu.sync_copy(dst_hbm.at[pl.ds(w * TPW, TPW)], dbuf)
    lanes = lax.iota(jnp.int32, 16)
    zero16 = jnp.zeros((16,), jnp.int32)

    def body(i, carry):
        c0, c1 = carry
        d = dbuf[pl.ds(i * 16, 16)]
        r = lax.shift_right_logical(d, RSH)
        for rr in range(NR):
            mi = jnp.where(r == rr, 1, 0).astype(jnp.int32)
            pc = _vsum_splat(mi)
            if rr < 16:
                c0 = c0 + jnp.where(lanes == rr, pc, 0)
            else:
                c1 = c1 + jnp.where(lanes == rr - 16, pc, 0)
        return c0, c1

    c0, c1 = lax.fori_loop(0, NV, body, (zero16, zero16))
    cbuf[pl.ds(0, 16)] = c0
    cbuf[pl.ds(16, 16)] = c1
    pltpu.sync_copy(cbuf, counts_hbm.at[w])


@functools.partial(
    pl.kernel,
    out_type=[
        jax.ShapeDtypeStruct((NR * CAP,), jnp.int32),
        jax.ShapeDtypeStruct((NR * CAP,), jnp.int32),
    ],
    mesh=_MESH,
    scratch_types=[
        pltpu.VMEM((TPW,), jnp.int32),
        pltpu.VMEM((TPW,), jnp.int32),
        pltpu.VMEM((TPW,), jnp.int32),
        pltpu.VMEM((TPW,), jnp.int32),
        pltpu.VMEM((32,), jnp.int32),
    ],
)
def _b2_place(dst_hbm, src_hbm, offs_hbm, srcb_hbm, dstob_hbm,
              dbuf, sbuf, posb, dlob, obuf):
    w = lax.axis_index("s") * 2 + lax.axis_index("c")
    base = w * TPW
    pltpu.sync_copy(dst_hbm.at[pl.ds(base, TPW)], dbuf)
    pltpu.sync_copy(src_hbm.at[pl.ds(base, TPW)], sbuf)
    pltpu.sync_copy(offs_hbm.at[w], obuf)
    o0 = obuf[pl.ds(0, 16)]
    o1 = obuf[pl.ds(16, 16)]
    offs0 = tuple(o0[rr] if rr < 16 else o1[rr - 16] for rr in range(NR))

    def body(i, offs):
        offs = list(offs)
        d = dbuf[pl.ds(i * 16, 16)]
        r = lax.shift_right_logical(d, RSH)
        pos = jnp.zeros((16,), jnp.int32)
        for rr in range(NR):
            m = r == rr
            mi = jnp.where(m, 1, 0).astype(jnp.int32)
            incl = _incl_cumsum(mi)
            pos = jnp.where(m, offs[rr] + incl - mi, pos)
            offs[rr] = offs[rr] + incl[15]
        posb[pl.ds(i * 16, 16)] = pos
        dlob[pl.ds(i * 16, 16)] = jnp.bitwise_and(d, RSZ - 1)
        return tuple(offs)

    lax.fori_loop(0, NV, body, offs0)
    pltpu.sync_copy(sbuf, srcb_hbm.at[posb])
    pltpu.sync_copy(dlob, dstob_hbm.at[posb])


_RND_SCRATCH = (
    [pltpu.VMEM_SHARED((ACC, H), jnp.float32)] * 2 +
    [pltpu.VMEM((ZR, H), jnp.float32)] +
    [pltpu.VMEM((CH, H), jnp.float32)] * 8 +
    [pltpu.VMEM((TRG,), jnp.int32)] * 2 +
    [pltpu.VMEM((NCH, CH), jnp.int32)] +
    [pltpu.VMEM((32,), jnp.int32)] +
    [pltpu.SemaphoreType.DMA] * 4
)


@functools.partial(
    pl.kernel,
    out_type=[
        jax.ShapeDtypeStruct((NP, H), jnp.float32),
        jax.ShapeDtypeStruct((NP, H), jnp.float32),
    ],
    mesh=_MESH,
    scratch_types=_RND_SCRATCH,
)
def _sc_round(h_hbm, c_hbm, huf_hbm, xwf_hbm, srcb_hbm, dstob_hbm, tot_hbm,
              hsum_hbm, csum_hbm,
              acc_h, acc_c, zbuf,
              rh0, rc0, rhuf0, rx0, rh1, rc1, rhuf1, rx1,
              sidx, didx, dlob, tbuf,
              lsem, zsem, gsem0, gsem1):
    cid = lax.axis_index("c")
    sid = lax.axis_index("s")
    lanes = lax.iota(jnp.int32, 16)
    pltpu.sync_copy(tot_hbm, tbuf)
    t0 = tbuf[pl.ds(0, 16)]
    t1 = tbuf[pl.ds(16, 16)]

    bufs = [
        (rh0, rc0, rhuf0, rx0, gsem0),
        (rh1, rc1, rhuf1, rx1, gsem1),
    ]

    zrows = ACC // 16

    def zrow(i, _):
        row = i // 8
        col = (i - row * 8) * 16
        zbuf[row, pl.ds(col, 16)] = jnp.zeros((16,), jnp.float32)
        return 0

    lax.fori_loop(0, ZR * 8, zrow, 0)

    def start_chunk(j, r):
        rh, rc, rhuf, rx, gsem = bufs[j % 2]
        sl = sidx.at[pl.ds(j * CH, CH)]
        dl = didx.at[pl.ds(j * CH, CH)]
        return (pltpu.async_copy(h_hbm.at[sl], rh, gsem),
                pltpu.async_copy(c_hbm.at[sl], rc, gsem),
                pltpu.async_copy(huf_hbm.at[sl], rhuf, gsem),
                pltpu.async_copy(xwf_hbm.at[dl], rx, gsem))

    def finish_chunk(j, cps):
        rh, rc, rhuf, rx, _ = bufs[j % 2]
        for cp in cps:
            cp.wait()

        def fc_body(q, _):
            row = q // 8
            col = (q - row * 8) * 16
            xv = rx[row, pl.ds(col, 16)]
            hv = rhuf[row, pl.ds(col, 16)]
            cv = rc[row, pl.ds(col, 16)]
            f = 1.0 / (1.0 + jnp.exp(-(xv + hv)))
            rx[row, pl.ds(col, 16)] = f * cv
            return 0

        lax.fori_loop(0, CH * 8, fc_body, 0)
        dl = dlob.at[j]
        pltpu.sync_copy(rh, acc_h.at[dl], add=True)
        pltpu.sync_copy(rx, acc_c.at[dl], add=True)

    def range_body(k, _):
        r = 2 * k + cid

        @pl.when(r < NR)
        def _():
            tv = (jnp.where(lanes == r, t0, 0) +
                  jnp.where(lanes == r - 16, t1, 0))
            total = _vsum_splat(tv)
            pos0 = r * CAP + sid * TRG
            zcp = []
            for z in range(zrows // ZR):
                rows = pl.ds(sid * zrows + z * ZR, ZR)
                zcp.append(pltpu.async_copy(zbuf, acc_h.at[rows], zsem))
                zcp.append(pltpu.async_copy(zbuf, acc_c.at[rows], zsem))
            pend = [pltpu.async_copy(
                srcb_hbm.at[pl.ds(pos0, TRG)], sidx, lsem)]
            for j in range(NCH):
                pend.append(pltpu.async_copy(
                    dstob_hbm.at[pl.ds(pos0 + j * CH, CH)], dlob.at[j],
                    lsem))
            for cp in pend:
                cp.wait()

            for j in range(NCH):
                for m in range(CH // 16):
                    off = j * CH + m * 16
                    s = sidx[pl.ds(off, 16)]
                    d = dlob[j, pl.ds(m * 16, 16)]
                    gpos = sid * TRG + off + lanes
                    valid = gpos < total
                    spread = jnp.bitwise_and(gpos, 63)
                    s = jnp.where(valid, s, spread)
                    d = jnp.where(valid, d, RSZ + spread)
                    sidx[pl.ds(off, 16)] = s
                    dlob[j, pl.ds(m * 16, 16)] = d
                    didx[pl.ds(off, 16)] = jnp.minimum(
                        r * RSZ + d, NP - 1)

            for cp in zcp:
                cp.wait()
            plsc.subcore_barrier()
            cps = start_chunk(0, r)
            for j in range(NCH):
                nxt = (start_chunk(j + 1, r) if j + 1 < NCH else None)
                finish_chunk(j, cps)
                cps = nxt
            plsc.subcore_barrier()
            wr = RSZ // 16
            wb = (pltpu.async_copy(
                      acc_h.at[pl.ds(sid * wr, wr)],
                      hsum_hbm.at[pl.ds(r * RSZ + sid * wr, wr)], lsem),
                  pltpu.async_copy(
                      acc_c.at[pl.ds(sid * wr, wr)],
                      csum_hbm.at[pl.ds(r * RSZ + sid * wr, wr)], lsem))
            for cp in wb:
                cp.wait()
            plsc.subcore_barrier()

        return 0

    lax.fori_loop(0, 13, range_body, 0)


def _tree(wordid, edge_index, emb, W_iou, U_iou, b_iou, W_f, U_f, b_f):
    pad_n = NP - N
    wid = jnp.concatenate(
        [wordid.astype(jnp.int32),
         (jnp.arange(pad_n, dtype=jnp.int32) & 63)])
    x = _sc_gather(emb, wid, 320)
    src = edge_index[0].astype(jnp.int32)
    dst = edge_index[1].astype(jnp.int32)
    pad_e = EP - E
    src_p = jnp.concatenate(
        [src, (jnp.arange(pad_e, dtype=jnp.int32) & 63)])
    dst_p = jnp.concatenate(
        [dst, jnp.full((pad_e,), NP - 1, jnp.int32)])
    counts = _b1_count(dst_p)
    offs, totals = _offsets(counts)
    srcb, dstob = _b2_place(dst_p, src_p, offs)
    ioux, xwf, h, c, huf = _node_init(x, W_iou, b_iou, W_f, b_f, U_f)
    for k in range(3):
        hsum, csum = _sc_round(h, c, huf, xwf, srcb, dstob, totals)
        h, c, huf = _round_update(ioux, hsum, csum, U_iou, U_f,
                                  last=(k == 2))
    return h


def kernel(wordid_a, edge_index_a, root_ids_a, wordid_b, edge_index_b,
           root_ids_b, emb, W_iou, U_iou, b_iou, W_f, U_f, b_f, wh_W, wh_b,
           wp_W, wp_b, r):
    h_a = _tree(wordid_a, edge_index_a, emb, W_iou, U_iou, b_iou, W_f,
                U_f, b_f)
    h_b = _tree(wordid_b, edge_index_b, emb, W_iou, U_iou, b_iou, W_f,
                U_f, b_f)
    ha = _sc_gather(h_a, root_ids_a.astype(jnp.int32), 16)
    hb = _sc_gather(h_b, root_ids_b.astype(jnp.int32), 16)
    out, pred = _final_mlp(ha, hb, wh_W, wh_b, wp_W, wp_b, r)
    return (out, pred)
